# baseline (speedup 1.0000x reference)
import functools

import numpy as np
import jax
import jax.numpy as jnp
from jax import lax
from jax.experimental import pallas as pl
from jax.experimental.pallas import tpu as pltpu
from jax.experimental.pallas import tpu_sc as plsc

N = 10000
E = 160000
DF = 128
DE = 16
H1 = 32
H2 = 16

NC = 2
NS = 16
NW = NC * NS
CHUNK = 128
CPW = 40
E_PAD = NW * CPW * CHUNK
N_PAD = 10240
SLAB = N_PAD // NS

@functools.lru_cache(None)
def _sc_mesh():
  return plsc.VectorSubcoreMesh(
      core_axis_name="c", subcore_axis_name="s", num_cores=NC, num_subcores=NS)


NB = 4
IPW = CPW // NB


def _make_gather(D):
  def body(table_hbm, idx_hbm, out_hbm, idx_v, r0, r1, r2, r3,
           g0, g1, g2, g3, w0, w1, w2, w3):
    rows = [r0, r1, r2, r3]
    gs = [g0, g1, g2, g3]
    ws = [w0, w1, w2, w3]
    wid = lax.axis_index("s") * NC + lax.axis_index("c")
    base = wid * CPW * CHUNK
    pltpu.sync_copy(idx_hbm.at[pl.ds(base, CPW * CHUNK)], idx_v)

    def outer(j4, carry):
      for b in range(NB):
        j = j4 * NB + b

        @pl.when(j4 > 0)
        def _():
          pltpu.make_async_copy(
              rows[b], out_hbm.at[pl.ds(base, CHUNK)], ws[b]).wait()

        pltpu.async_copy(
            table_hbm.at[idx_v.at[pl.ds(j * CHUNK, CHUNK)]], rows[b], gs[b])
      for b in range(NB):
        j = j4 * NB + b
        pltpu.make_async_copy(
            table_hbm.at[idx_v.at[pl.ds(j * CHUNK, CHUNK)]], rows[b],
            gs[b]).wait()
        pltpu.async_copy(rows[b], out_hbm.at[pl.ds(base + j * CHUNK, CHUNK)],
                         ws[b])
      return carry

    lax.fori_loop(0, IPW, outer, 0)
    for b in range(NB):
      pltpu.make_async_copy(
          rows[b], out_hbm.at[pl.ds(base, CHUNK)], ws[b]).wait()

  params = (pltpu.CompilerParams() if D % 128 == 0 else
            pltpu.CompilerParams(use_tc_tiling_on_sc=False))
  return pl.kernel(
      body,
      out_type=jax.ShapeDtypeStruct((E_PAD, D), jnp.float32),
      mesh=_sc_mesh(),
      compiler_params=params,
      scratch_types=(
          [pltpu.VMEM((CPW * CHUNK,), jnp.int32)]
          + [pltpu.VMEM((CHUNK, D), jnp.float32)] * NB
          + [pltpu.SemaphoreType.DMA] * (2 * NB)
      ),
  )


def _make_scatter(D):
  def body(msg_hbm, dst2_hbm, zeros_hbm, out_hbm, idx_v, m0, m1, m2, m3,
           aggr, s0, s1, s2, s3):
    msgs = [m0, m1, m2, m3]
    ms = [s0, s1, s2, s3]
    cid = lax.axis_index("c")
    sid = lax.axis_index("s")
    wid = sid * NC + cid
    pltpu.sync_copy(zeros_hbm.at[pl.ds(sid * SLAB, SLAB)],
                    aggr.at[pl.ds(sid * SLAB, SLAB)])
    pltpu.sync_copy(dst2_hbm.at[pl.ds(wid * CPW, CPW)], idx_v)
    plsc.subcore_barrier()
    base = wid * CPW * CHUNK
    for b in range(NB):
      pltpu.async_copy(msg_hbm.at[pl.ds(base + b * CHUNK, CHUNK)], msgs[b],
                       ms[b])

    def outer(j4, carry):
      for b in range(NB):
        j = j4 * NB + b
        pltpu.make_async_copy(
            msg_hbm.at[pl.ds(base, CHUNK)], msgs[b], ms[b]).wait()
        pltpu.sync_copy(msgs[b], aggr.at[idx_v.at[j]], add=True)

        @pl.when(j4 < IPW - 1)
        def _():
          pltpu.async_copy(
              msg_hbm.at[pl.ds(base + (j + NB) * CHUNK, CHUNK)], msgs[b],
              ms[b])
      return carry

    lax.fori_loop(0, IPW, outer, 0)
    plsc.subcore_barrier()
    out_off = cid * N_PAD + sid * SLAB
    pltpu.sync_copy(aggr.at[pl.ds(sid * SLAB, SLAB)],
                    out_hbm.at[pl.ds(out_off, SLAB)])

  return pl.kernel(
      body,
      out_type=jax.ShapeDtypeStruct((NC * N_PAD, D), jnp.float32),
      mesh=_sc_mesh(),
      compiler_params=pltpu.CompilerParams(use_tc_tiling_on_sc=False),
      scratch_types=(
          [pltpu.VMEM((CPW, CHUNK), jnp.int32)]
          + [pltpu.VMEM((CHUNK, D), jnp.float32)] * NB
          + [pltpu.VMEM_SHARED((N_PAD, D), jnp.float32)]
          + [pltpu.SemaphoreType.DMA] * NB
      ),
  )


_make_gather = functools.lru_cache(None)(_make_gather)
_make_scatter = functools.lru_cache(None)(_make_scatter)


def _msg_body(xs_ref, ea_ref, Wa_ref, ba_ref, M_ref, R_ref, u_ref, S_ref,
              out_ref):
  f32 = jnp.float32
  bf16 = jnp.bfloat16
  h = jnp.maximum(
      jnp.dot(ea_ref[...], Wa_ref[...], preferred_element_type=f32)
      + ba_ref[...], 0.0)
  T = jnp.dot(xs_ref[...].astype(bf16), M_ref[...].astype(bf16),
              preferred_element_type=f32)
  hrep = jnp.dot(h, R_ref[...], preferred_element_type=f32) + u_ref[...]
  out_ref[...] = jnp.dot((T * hrep).astype(bf16), S_ref[...].astype(bf16),
                         preferred_element_type=f32)


def _msg_call(xs, ea_p, Wa, ba, M, R, u, S, cout, be=1024):
  cin = xs.shape[1]
  dw = M.shape[1]
  return pl.pallas_call(
      _msg_body,
      grid=(E_PAD // be,),
      in_specs=[
          pl.BlockSpec((be, cin), lambda i: (i, 0)),
          pl.BlockSpec((be, DE), lambda i: (i, 0)),
          pl.BlockSpec((DE, 16), lambda i: (0, 0)),
          pl.BlockSpec((1, 16), lambda i: (0, 0)),
          pl.BlockSpec((cin, dw), lambda i: (0, 0)),
          pl.BlockSpec((16, dw), lambda i: (0, 0)),
          pl.BlockSpec((1, dw), lambda i: (0, 0)),
          pl.BlockSpec((dw, cout), lambda i: (0, 0)),
      ],
      out_specs=pl.BlockSpec((be, cout), lambda i: (i, 0)),
      out_shape=jax.ShapeDtypeStruct((E_PAD, cout), jnp.float32),
  )(xs, ea_p, Wa, ba, M, R, u, S)


def _node1_body(p0_ref, p1_ref, x_ref, root_ref, bias_ref, out_ref):
  agg = p0_ref[...] + p1_ref[...]
  out_ref[...] = jnp.maximum(
      agg + jnp.dot(x_ref[...], root_ref[...],
                    preferred_element_type=jnp.float32) + bias_ref[...], 0.0)


def _node1_call(p0, p1, x, root, bias, bn=2000):
  cin = x.shape[1]
  cout = root.shape[1]
  return pl.pallas_call(
      _node1_body,
      grid=(N // bn,),
      in_specs=[
          pl.BlockSpec((bn, cout), lambda i: (i, 0)),
          pl.BlockSpec((bn, cout), lambda i: (i, 0)),
          pl.BlockSpec((bn, cin), lambda i: (i, 0)),
          pl.BlockSpec((cin, cout), lambda i: (0, 0)),
          pl.BlockSpec((1, cout), lambda i: (0, 0)),
      ],
      out_specs=pl.BlockSpec((bn, cout), lambda i: (i, 0)),
      out_shape=jax.ShapeDtypeStruct((N, cout), jnp.float32),
  )(p0, p1, x, root, bias)


def _node2_body(p0_ref, p1_ref, y1_ref, root_ref, bias_ref, w1_ref, c1_ref,
                w2_ref, c2_ref, out_ref):
  f32 = jnp.float32
  agg = p0_ref[...] + p1_ref[...]
  y2 = jnp.maximum(
      agg + jnp.dot(y1_ref[...], root_ref[...], preferred_element_type=f32)
      + bias_ref[...], 0.0)
  h3 = jnp.maximum(
      jnp.dot(y2, w1_ref[...], preferred_element_type=f32) + c1_ref[...], 0.0)
  out_ref[...] = jnp.dot(h3, w2_ref[...],
                         preferred_element_type=f32) + c2_ref[...]


def _node2_call(p0, p1, y1, root, bias, w1, c1, w2, c2, bn=2000):
  return pl.pallas_call(
      _node2_body,
      grid=(N // bn,),
      in_specs=[
          pl.BlockSpec((bn, H2), lambda i: (i, 0)),
          pl.BlockSpec((bn, H2), lambda i: (i, 0)),
          pl.BlockSpec((bn, H1), lambda i: (i, 0)),
          pl.BlockSpec((H1, H2), lambda i: (0, 0)),
          pl.BlockSpec((1, H2), lambda i: (0, 0)),
          pl.BlockSpec((H2, 8), lambda i: (0, 0)),
          pl.BlockSpec((1, 8), lambda i: (0, 0)),
          pl.BlockSpec((8, 1), lambda i: (0, 0)),
          pl.BlockSpec((1, 1), lambda i: (0, 0)),
      ],
      out_specs=pl.BlockSpec((bn, 1), lambda i: (i, 0)),
      out_shape=jax.ShapeDtypeStruct((N, 1), jnp.float32),
  )(p0, p1, y1, root, bias, w1, c1, w2, c2)


def _repmat(cout):
  r = np.zeros((16, 17 * cout), np.float32)
  for k in range(16):
    r[k, k * cout:(k + 1) * cout] = 1.0
  u = np.concatenate(
      [np.zeros((16 * cout,), np.float32), np.ones((cout,), np.float32)])
  s = np.tile(np.eye(cout, dtype=np.float32), (17, 1))
  return r, u.reshape(1, -1), s


_R1, _U1, _S1 = _repmat(H1)
_R2, _U2, _S2 = _repmat(H2)


def kernel(x, edge_index, edge_attr, W1a, b1a, W1b, b1b, root1, bias1,
           W2a, b2a, W2b, b2b, root2, bias2, lin1_w, lin1_b, lin2_w, lin2_b):
  src = edge_index[0]
  dst = edge_index[1]
  pad = E_PAD - E
  src_p = jnp.concatenate([src, jnp.zeros((pad,), jnp.int32)])
  dst_p = jnp.concatenate([dst, jnp.full((pad,), N, jnp.int32)])
  ea_p = jnp.concatenate([edge_attr, jnp.zeros((pad, DE), jnp.float32)])

  M1 = jnp.concatenate([
      W1b.reshape(16, DF, H1).transpose(1, 0, 2).reshape(DF, 16 * H1),
      b1b.reshape(DF, H1)], axis=1)
  M2 = jnp.concatenate([
      W2b.reshape(16, H1, H2).transpose(1, 0, 2).reshape(H1, 16 * H2),
      b2b.reshape(H1, H2)], axis=1)

  z1 = jnp.zeros((N_PAD, H1), jnp.float32)
  z2 = jnp.zeros((N_PAD, H2), jnp.float32)

  xs = _make_gather(DF)(x, src_p)
  msg1 = _msg_call(xs, ea_p, W1a, b1a.reshape(1, 16), M1, _R1, _U1, _S1, H1)
  dst2 = dst_p.reshape(E_PAD // CHUNK, CHUNK)
  p1 = _make_scatter(H1)(msg1, dst2, z1)
  y1 = _node1_call(p1[:N], p1[N_PAD:N_PAD + N], x, root1,
                   bias1.reshape(1, H1))

  ys = _make_gather(H1)(y1, src_p)
  msg2 = _msg_call(ys, ea_p, W2a, b2a.reshape(1, 16), M2, _R2, _U2, _S2, H2)
  p2 = _make_scatter(H2)(msg2, dst2, z2)
  out = _node2_call(p2[:N], p2[N_PAD:N_PAD + N], y1, root2,
                    bias2.reshape(1, H2), lin1_w, lin1_b.reshape(1, 8),
                    lin2_w, lin2_b.reshape(1, 1))
  return out


# speedup vs baseline: 1.0363x; 1.0363x over previous
import functools

import numpy as np
import jax
import jax.numpy as jnp
from jax import lax
from jax.experimental import pallas as pl
from jax.experimental.pallas import tpu as pltpu
from jax.experimental.pallas import tpu_sc as plsc

N = 10000
E = 160000
DF = 128
DE = 16
H1 = 32
H2 = 16

NC = 2
NS = 16
NW = NC * NS
CHUNK = 128
CPW = 40
E_PAD = NW * CPW * CHUNK
N_PAD = 10240
SLAB = N_PAD // NS

@functools.lru_cache(None)
def _sc_mesh():
  return plsc.VectorSubcoreMesh(
      core_axis_name="c", subcore_axis_name="s", num_cores=NC, num_subcores=NS)


NB = 4
IPW = CPW // NB


def _make_gather(D):
  def body(table_hbm, idx_hbm, out_hbm, idx_v, r0, r1, r2, r3,
           g0, g1, g2, g3, w0, w1, w2, w3):
    rows = [r0, r1, r2, r3]
    gs = [g0, g1, g2, g3]
    ws = [w0, w1, w2, w3]
    wid = lax.axis_index("s") * NC + lax.axis_index("c")
    base = wid * CPW * CHUNK
    pltpu.sync_copy(idx_hbm.at[pl.ds(base, CPW * CHUNK)], idx_v)

    def outer(j4, carry):
      for b in range(NB):
        j = j4 * NB + b

        @pl.when(j4 > 0)
        def _():
          pltpu.make_async_copy(
              rows[b], out_hbm.at[pl.ds(base, CHUNK)], ws[b]).wait()

        pltpu.async_copy(
            table_hbm.at[idx_v.at[pl.ds(j * CHUNK, CHUNK)]], rows[b], gs[b])
      for b in range(NB):
        j = j4 * NB + b
        pltpu.make_async_copy(
            table_hbm.at[idx_v.at[pl.ds(j * CHUNK, CHUNK)]], rows[b],
            gs[b]).wait()
        pltpu.async_copy(rows[b], out_hbm.at[pl.ds(base + j * CHUNK, CHUNK)],
                         ws[b])
      return carry

    lax.fori_loop(0, IPW, outer, 0)
    for b in range(NB):
      pltpu.make_async_copy(
          rows[b], out_hbm.at[pl.ds(base, CHUNK)], ws[b]).wait()

  params = (pltpu.CompilerParams() if D % 128 == 0 else
            pltpu.CompilerParams(use_tc_tiling_on_sc=False))
  return pl.kernel(
      body,
      out_type=jax.ShapeDtypeStruct((E_PAD, D), jnp.float32),
      mesh=_sc_mesh(),
      compiler_params=params,
      scratch_types=(
          [pltpu.VMEM((CPW * CHUNK,), jnp.int32)]
          + [pltpu.VMEM((CHUNK, D), jnp.float32)] * NB
          + [pltpu.SemaphoreType.DMA] * (2 * NB)
      ),
  )


def _make_scatter(D):
  def body(msg_hbm, dst2_hbm, zeros_hbm, out_hbm, idx_v, m0, m1, m2, m3,
           aggr, s0, s1, s2, s3):
    msgs = [m0, m1, m2, m3]
    ms = [s0, s1, s2, s3]
    cid = lax.axis_index("c")
    sid = lax.axis_index("s")
    wid = sid * NC + cid
    pltpu.sync_copy(zeros_hbm.at[pl.ds(sid * SLAB, SLAB)],
                    aggr.at[pl.ds(sid * SLAB, SLAB)])
    pltpu.sync_copy(dst2_hbm.at[pl.ds(wid * CPW, CPW)], idx_v)
    plsc.subcore_barrier()
    base = wid * CPW * CHUNK
    for b in range(NB):
      pltpu.async_copy(msg_hbm.at[pl.ds(base + b * CHUNK, CHUNK)], msgs[b],
                       ms[b])

    def outer(j4, carry):
      for b in range(NB):
        j = j4 * NB + b
        pltpu.make_async_copy(
            msg_hbm.at[pl.ds(base, CHUNK)], msgs[b], ms[b]).wait()
        pltpu.sync_copy(msgs[b], aggr.at[idx_v.at[j]], add=True)

        @pl.when(j4 < IPW - 1)
        def _():
          pltpu.async_copy(
              msg_hbm.at[pl.ds(base + (j + NB) * CHUNK, CHUNK)], msgs[b],
              ms[b])
      return carry

    lax.fori_loop(0, IPW, outer, 0)
    plsc.subcore_barrier()
    out_off = cid * N_PAD + sid * SLAB
    pltpu.sync_copy(aggr.at[pl.ds(sid * SLAB, SLAB)],
                    out_hbm.at[pl.ds(out_off, SLAB)])

  return pl.kernel(
      body,
      out_type=jax.ShapeDtypeStruct((NC * N_PAD, D), jnp.float32),
      mesh=_sc_mesh(),
      compiler_params=pltpu.CompilerParams(use_tc_tiling_on_sc=False),
      scratch_types=(
          [pltpu.VMEM((CPW, CHUNK), jnp.int32)]
          + [pltpu.VMEM((CHUNK, D), jnp.float32)] * NB
          + [pltpu.VMEM_SHARED((N_PAD, D), jnp.float32)]
          + [pltpu.SemaphoreType.DMA] * NB
      ),
  )


_make_gather = functools.lru_cache(None)(_make_gather)
_make_scatter = functools.lru_cache(None)(_make_scatter)


def _msg_body(xs_ref, ea_ref, Wa_ref, ba_ref, M_ref, R_ref, u_ref, S_ref,
              out_ref):
  f32 = jnp.float32
  bf16 = jnp.bfloat16
  h = jnp.maximum(
      jnp.dot(ea_ref[...], Wa_ref[...], preferred_element_type=f32)
      + ba_ref[...], 0.0)
  T = jnp.dot(xs_ref[...].astype(bf16), M_ref[...].astype(bf16),
              preferred_element_type=f32)
  hrep = jnp.dot(h.astype(bf16), R_ref[...].astype(bf16),
                 preferred_element_type=f32) + u_ref[...]
  out_ref[...] = jnp.dot((T * hrep).astype(bf16), S_ref[...].astype(bf16),
                         preferred_element_type=f32)


def _msg_call(xs, ea, Wa, ba, M, R, u, S, cout, be=1280):
  cin = xs.shape[1]
  dw = M.shape[1]
  last = E // be - 1
  return pl.pallas_call(
      _msg_body,
      grid=(E_PAD // be,),
      in_specs=[
          pl.BlockSpec((be, cin), lambda i: (i, 0)),
          pl.BlockSpec((be, DE), lambda i: (jnp.minimum(i, last), 0)),
          pl.BlockSpec((DE, 128), lambda i: (0, 0)),
          pl.BlockSpec((1, 128), lambda i: (0, 0)),
          pl.BlockSpec((cin, dw), lambda i: (0, 0)),
          pl.BlockSpec((128, dw), lambda i: (0, 0)),
          pl.BlockSpec((1, dw), lambda i: (0, 0)),
          pl.BlockSpec((dw, cout), lambda i: (0, 0)),
      ],
      out_specs=pl.BlockSpec((be, cout), lambda i: (i, 0)),
      out_shape=jax.ShapeDtypeStruct((E_PAD, cout), jnp.float32),
  )(xs, ea, Wa, ba, M, R, u, S)


def _node1_body(p0_ref, p1_ref, x_ref, root_ref, bias_ref, out_ref):
  agg = p0_ref[...] + p1_ref[...]
  out_ref[...] = jnp.maximum(
      agg + jnp.dot(x_ref[...], root_ref[...],
                    preferred_element_type=jnp.float32) + bias_ref[...], 0.0)


def _node1_call(p0, p1, x, root, bias, bn=2000):
  cin = x.shape[1]
  cout = root.shape[1]
  return pl.pallas_call(
      _node1_body,
      grid=(N // bn,),
      in_specs=[
          pl.BlockSpec((bn, cout), lambda i: (i, 0)),
          pl.BlockSpec((bn, cout), lambda i: (i, 0)),
          pl.BlockSpec((bn, cin), lambda i: (i, 0)),
          pl.BlockSpec((cin, cout), lambda i: (0, 0)),
          pl.BlockSpec((1, cout), lambda i: (0, 0)),
      ],
      out_specs=pl.BlockSpec((bn, cout), lambda i: (i, 0)),
      out_shape=jax.ShapeDtypeStruct((N, cout), jnp.float32),
  )(p0, p1, x, root, bias)


def _node2_body(p0_ref, p1_ref, y1_ref, root_ref, bias_ref, w1_ref, c1_ref,
                w2_ref, c2_ref, out_ref):
  f32 = jnp.float32
  agg = p0_ref[...] + p1_ref[...]
  y2 = jnp.maximum(
      agg + jnp.dot(y1_ref[...], root_ref[...], preferred_element_type=f32)
      + bias_ref[...], 0.0)
  h3 = jnp.maximum(
      jnp.dot(y2, w1_ref[...], preferred_element_type=f32) + c1_ref[...], 0.0)
  out_ref[...] = jnp.dot(h3, w2_ref[...],
                         preferred_element_type=f32) + c2_ref[...]


def _node2_call(p0, p1, y1, root, bias, w1, c1, w2, c2, bn=2000):
  return pl.pallas_call(
      _node2_body,
      grid=(N // bn,),
      in_specs=[
          pl.BlockSpec((bn, H2), lambda i: (i, 0)),
          pl.BlockSpec((bn, H2), lambda i: (i, 0)),
          pl.BlockSpec((bn, H1), lambda i: (i, 0)),
          pl.BlockSpec((H1, H2), lambda i: (0, 0)),
          pl.BlockSpec((1, H2), lambda i: (0, 0)),
          pl.BlockSpec((H2, 8), lambda i: (0, 0)),
          pl.BlockSpec((1, 8), lambda i: (0, 0)),
          pl.BlockSpec((8, 1), lambda i: (0, 0)),
          pl.BlockSpec((1, 1), lambda i: (0, 0)),
      ],
      out_specs=pl.BlockSpec((bn, 1), lambda i: (i, 0)),
      out_shape=jax.ShapeDtypeStruct((N, 1), jnp.float32),
  )(p0, p1, y1, root, bias, w1, c1, w2, c2)


def _repmat(cout):
  r = np.zeros((128, 17 * cout), np.float32)
  for k in range(16):
    r[k, k * cout:(k + 1) * cout] = 1.0
  u = np.concatenate(
      [np.zeros((16 * cout,), np.float32), np.ones((cout,), np.float32)])
  s = np.tile(np.eye(cout, dtype=np.float32), (17, 1))
  return r, u.reshape(1, -1), s


_R1, _U1, _S1 = _repmat(H1)
_R2, _U2, _S2 = _repmat(H2)


def kernel(x, edge_index, edge_attr, W1a, b1a, W1b, b1b, root1, bias1,
           W2a, b2a, W2b, b2b, root2, bias2, lin1_w, lin1_b, lin2_w, lin2_b):
  src = edge_index[0]
  dst = edge_index[1]
  pad = E_PAD - E
  src_p = jnp.concatenate([src, jnp.zeros((pad,), jnp.int32)])
  dst_p = jnp.concatenate([dst, jnp.full((pad,), N, jnp.int32)])

  M1 = jnp.concatenate([
      W1b.reshape(16, DF, H1).transpose(1, 0, 2).reshape(DF, 16 * H1),
      b1b.reshape(DF, H1)], axis=1)
  M2 = jnp.concatenate([
      W2b.reshape(16, H1, H2).transpose(1, 0, 2).reshape(H1, 16 * H2),
      b2b.reshape(H1, H2)], axis=1)

  z1 = jnp.zeros((N_PAD, H1), jnp.float32)
  z2 = jnp.zeros((N_PAD, H2), jnp.float32)

  xs = _make_gather(DF)(x, src_p)
  W1a_p = jnp.pad(W1a, ((0, 0), (0, 112)))
  b1a_p = jnp.pad(b1a, (0, 112)).reshape(1, 128)
  msg1 = _msg_call(xs, edge_attr, W1a_p, b1a_p, M1, _R1, _U1, _S1, H1)
  dst2 = dst_p.reshape(E_PAD // CHUNK, CHUNK)
  p1 = _make_scatter(H1)(msg1, dst2, z1)
  y1 = _node1_call(p1[:N], p1[N_PAD:N_PAD + N], x, root1,
                   bias1.reshape(1, H1))

  ys = _make_gather(H1)(y1, src_p)
  W2a_p = jnp.pad(W2a, ((0, 0), (0, 112)))
  b2a_p = jnp.pad(b2a, (0, 112)).reshape(1, 128)
  msg2 = _msg_call(ys, edge_attr, W2a_p, b2a_p, M2, _R2, _U2, _S2, H2)
  p2 = _make_scatter(H2)(msg2, dst2, z2)
  out = _node2_call(p2[:N], p2[N_PAD:N_PAD + N], y1, root2,
                    bias2.reshape(1, H2), lin1_w, lin1_b.reshape(1, 8),
                    lin2_w, lin2_b.reshape(1, 1))
  return out


# speedup vs baseline: 1.0422x; 1.0057x over previous
import functools

import numpy as np
import jax
import jax.numpy as jnp
from jax import lax
from jax.experimental import pallas as pl
from jax.experimental.pallas import tpu as pltpu
from jax.experimental.pallas import tpu_sc as plsc

N = 10000
E = 160000
DF = 128
DE = 16
H1 = 32
H2 = 16

NC = 2
NS = 16
NW = NC * NS
CHUNK = 128
CPW = 40
CPW0 = 60
CPW1 = 20
IDXPAD = CPW0 * CHUNK
E_PAD = NW * CPW * CHUNK
N_PAD = 10240
SLAB = N_PAD // NS

@functools.lru_cache(None)
def _sc_mesh():
  return plsc.VectorSubcoreMesh(
      core_axis_name="c", subcore_axis_name="s", num_cores=NC, num_subcores=NS)


NB = 4
IPW = CPW // NB


def _make_gather(D):
  def body(table_hbm, idx_hbm, out_hbm, idx_v, r0, r1, r2, r3,
           g0, g1, g2, g3, w0, w1, w2, w3):
    rows = [r0, r1, r2, r3]
    gs = [g0, g1, g2, g3]
    ws = [w0, w1, w2, w3]
    cid = lax.axis_index("c")
    sid = lax.axis_index("s")
    n4 = jnp.where(cid == 0, CPW0 // NB, CPW1 // NB)
    base = jnp.where(cid == 0, sid * CPW0, NS * CPW0 + sid * CPW1) * CHUNK
    pltpu.sync_copy(idx_hbm.at[pl.ds(base, CPW0 * CHUNK)], idx_v)

    def outer(j4, carry):
      for b in range(NB):
        j = j4 * NB + b

        @pl.when(j4 > 0)
        def _():
          pltpu.make_async_copy(
              rows[b], out_hbm.at[pl.ds(base, CHUNK)], ws[b]).wait()

        pltpu.async_copy(
            table_hbm.at[idx_v.at[pl.ds(j * CHUNK, CHUNK)]], rows[b], gs[b])
      for b in range(NB):
        j = j4 * NB + b
        pltpu.make_async_copy(
            table_hbm.at[idx_v.at[pl.ds(j * CHUNK, CHUNK)]], rows[b],
            gs[b]).wait()
        pltpu.async_copy(rows[b], out_hbm.at[pl.ds(base + j * CHUNK, CHUNK)],
                         ws[b])
      return carry

    lax.fori_loop(0, n4, outer, 0)
    for b in range(NB):
      pltpu.make_async_copy(
          rows[b], out_hbm.at[pl.ds(base, CHUNK)], ws[b]).wait()

  params = (pltpu.CompilerParams() if D % 128 == 0 else
            pltpu.CompilerParams(use_tc_tiling_on_sc=False))
  return pl.kernel(
      body,
      out_type=jax.ShapeDtypeStruct((E_PAD, D), jnp.float32),
      mesh=_sc_mesh(),
      compiler_params=params,
      scratch_types=(
          [pltpu.VMEM((CPW0 * CHUNK,), jnp.int32)]
          + [pltpu.VMEM((CHUNK, D), jnp.float32)] * NB
          + [pltpu.SemaphoreType.DMA] * (2 * NB)
      ),
  )


def _make_scatter(D):
  def body(msg_hbm, dst2_hbm, zeros_hbm, out_hbm, idx_v, m0, m1, m2, m3,
           aggr, s0, s1, s2, s3):
    msgs = [m0, m1, m2, m3]
    ms = [s0, s1, s2, s3]
    cid = lax.axis_index("c")
    sid = lax.axis_index("s")
    wid = sid * NC + cid
    pltpu.sync_copy(zeros_hbm.at[pl.ds(sid * SLAB, SLAB)],
                    aggr.at[pl.ds(sid * SLAB, SLAB)])
    pltpu.sync_copy(dst2_hbm.at[pl.ds(wid * CPW, CPW)], idx_v)
    plsc.subcore_barrier()
    base = wid * CPW * CHUNK
    for b in range(NB):
      pltpu.async_copy(msg_hbm.at[pl.ds(base + b * CHUNK, CHUNK)], msgs[b],
                       ms[b])

    def outer(j4, carry):
      for b in range(NB):
        j = j4 * NB + b
        pltpu.make_async_copy(
            msg_hbm.at[pl.ds(base, CHUNK)], msgs[b], ms[b]).wait()
        pltpu.sync_copy(msgs[b], aggr.at[idx_v.at[j]], add=True)

        @pl.when(j4 < IPW - 1)
        def _():
          pltpu.async_copy(
              msg_hbm.at[pl.ds(base + (j + NB) * CHUNK, CHUNK)], msgs[b],
              ms[b])
      return carry

    lax.fori_loop(0, IPW, outer, 0)
    plsc.subcore_barrier()
    out_off = cid * N_PAD + sid * SLAB
    pltpu.sync_copy(aggr.at[pl.ds(sid * SLAB, SLAB)],
                    out_hbm.at[pl.ds(out_off, SLAB)])

  return pl.kernel(
      body,
      out_type=jax.ShapeDtypeStruct((NC * N_PAD, D), jnp.float32),
      mesh=_sc_mesh(),
      compiler_params=pltpu.CompilerParams(use_tc_tiling_on_sc=False),
      scratch_types=(
          [pltpu.VMEM((CPW, CHUNK), jnp.int32)]
          + [pltpu.VMEM((CHUNK, D), jnp.float32)] * NB
          + [pltpu.VMEM_SHARED((N_PAD, D), jnp.float32)]
          + [pltpu.SemaphoreType.DMA] * NB
      ),
  )


_make_gather = functools.lru_cache(None)(_make_gather)
_make_scatter = functools.lru_cache(None)(_make_scatter)


def _msg_body(xs_ref, ea_ref, Wa_ref, ba_ref, M_ref, R_ref, u_ref, S_ref,
              out_ref):
  f32 = jnp.float32
  bf16 = jnp.bfloat16
  h = jnp.maximum(
      jnp.dot(ea_ref[...], Wa_ref[...], preferred_element_type=f32)
      + ba_ref[...], 0.0)
  T = jnp.dot(xs_ref[...].astype(bf16), M_ref[...].astype(bf16),
              preferred_element_type=f32)
  hrep = jnp.dot(h.astype(bf16), R_ref[...].astype(bf16),
                 preferred_element_type=f32) + u_ref[...]
  out_ref[...] = jnp.dot((T * hrep).astype(bf16), S_ref[...].astype(bf16),
                         preferred_element_type=f32)


def _msg_call(xs, ea, Wa, ba, M, R, u, S, cout, be=1280):
  cin = xs.shape[1]
  dw = M.shape[1]
  last = E // be - 1
  return pl.pallas_call(
      _msg_body,
      grid=(E_PAD // be,),
      in_specs=[
          pl.BlockSpec((be, cin), lambda i: (i, 0)),
          pl.BlockSpec((be, DE), lambda i: (jnp.minimum(i, last), 0)),
          pl.BlockSpec((DE, 128), lambda i: (0, 0)),
          pl.BlockSpec((1, 128), lambda i: (0, 0)),
          pl.BlockSpec((cin, dw), lambda i: (0, 0)),
          pl.BlockSpec((128, dw), lambda i: (0, 0)),
          pl.BlockSpec((1, dw), lambda i: (0, 0)),
          pl.BlockSpec((dw, cout), lambda i: (0, 0)),
      ],
      out_specs=pl.BlockSpec((be, cout), lambda i: (i, 0)),
      out_shape=jax.ShapeDtypeStruct((E_PAD, cout), jnp.float32),
  )(xs, ea, Wa, ba, M, R, u, S)


def _node1_body(p0_ref, p1_ref, x_ref, root_ref, bias_ref, out_ref):
  agg = p0_ref[...] + p1_ref[...]
  out_ref[...] = jnp.maximum(
      agg + jnp.dot(x_ref[...], root_ref[...],
                    preferred_element_type=jnp.float32) + bias_ref[...], 0.0)


def _node1_call(p0, p1, x, root, bias, bn=2000):
  cin = x.shape[1]
  cout = root.shape[1]
  return pl.pallas_call(
      _node1_body,
      grid=(N // bn,),
      in_specs=[
          pl.BlockSpec((bn, cout), lambda i: (i, 0)),
          pl.BlockSpec((bn, cout), lambda i: (i, 0)),
          pl.BlockSpec((bn, cin), lambda i: (i, 0)),
          pl.BlockSpec((cin, cout), lambda i: (0, 0)),
          pl.BlockSpec((1, cout), lambda i: (0, 0)),
      ],
      out_specs=pl.BlockSpec((bn, cout), lambda i: (i, 0)),
      out_shape=jax.ShapeDtypeStruct((N, cout), jnp.float32),
  )(p0, p1, x, root, bias)


def _node2_body(p0_ref, p1_ref, y1_ref, root_ref, bias_ref, w1_ref, c1_ref,
                w2_ref, c2_ref, out_ref):
  f32 = jnp.float32
  agg = p0_ref[...] + p1_ref[...]
  y2 = jnp.maximum(
      agg + jnp.dot(y1_ref[...], root_ref[...], preferred_element_type=f32)
      + bias_ref[...], 0.0)
  h3 = jnp.maximum(
      jnp.dot(y2, w1_ref[...], preferred_element_type=f32) + c1_ref[...], 0.0)
  out_ref[...] = jnp.dot(h3, w2_ref[...],
                         preferred_element_type=f32) + c2_ref[...]


def _node2_call(p0, p1, y1, root, bias, w1, c1, w2, c2, bn=2000):
  return pl.pallas_call(
      _node2_body,
      grid=(N // bn,),
      in_specs=[
          pl.BlockSpec((bn, H2), lambda i: (i, 0)),
          pl.BlockSpec((bn, H2), lambda i: (i, 0)),
          pl.BlockSpec((bn, H1), lambda i: (i, 0)),
          pl.BlockSpec((H1, H2), lambda i: (0, 0)),
          pl.BlockSpec((1, H2), lambda i: (0, 0)),
          pl.BlockSpec((H2, 8), lambda i: (0, 0)),
          pl.BlockSpec((1, 8), lambda i: (0, 0)),
          pl.BlockSpec((8, 1), lambda i: (0, 0)),
          pl.BlockSpec((1, 1), lambda i: (0, 0)),
      ],
      out_specs=pl.BlockSpec((bn, 1), lambda i: (i, 0)),
      out_shape=jax.ShapeDtypeStruct((N, 1), jnp.float32),
  )(p0, p1, y1, root, bias, w1, c1, w2, c2)


def _repmat(cout):
  r = np.zeros((128, 17 * cout), np.float32)
  for k in range(16):
    r[k, k * cout:(k + 1) * cout] = 1.0
  u = np.concatenate(
      [np.zeros((16 * cout,), np.float32), np.ones((cout,), np.float32)])
  s = np.tile(np.eye(cout, dtype=np.float32), (17, 1))
  return r, u.reshape(1, -1), s


_R1, _U1, _S1 = _repmat(H1)
_R2, _U2, _S2 = _repmat(H2)


def kernel(x, edge_index, edge_attr, W1a, b1a, W1b, b1b, root1, bias1,
           W2a, b2a, W2b, b2b, root2, bias2, lin1_w, lin1_b, lin2_w, lin2_b):
  src = edge_index[0]
  dst = edge_index[1]
  pad = E_PAD - E
  src_p = jnp.concatenate([src, jnp.zeros((pad + IDXPAD,), jnp.int32)])
  dst_p = jnp.concatenate([dst, jnp.full((pad,), N, jnp.int32)])

  M1 = jnp.concatenate([
      W1b.reshape(16, DF, H1).transpose(1, 0, 2).reshape(DF, 16 * H1),
      b1b.reshape(DF, H1)], axis=1)
  M2 = jnp.concatenate([
      W2b.reshape(16, H1, H2).transpose(1, 0, 2).reshape(H1, 16 * H2),
      b2b.reshape(H1, H2)], axis=1)

  z1 = jnp.zeros((N_PAD, H1), jnp.float32)
  z2 = jnp.zeros((N_PAD, H2), jnp.float32)

  xs = _make_gather(DF)(x, src_p)
  W1a_p = jnp.pad(W1a, ((0, 0), (0, 112)))
  b1a_p = jnp.pad(b1a, (0, 112)).reshape(1, 128)
  msg1 = _msg_call(xs, edge_attr, W1a_p, b1a_p, M1, _R1, _U1, _S1, H1)
  dst2 = dst_p.reshape(E_PAD // CHUNK, CHUNK)
  p1 = _make_scatter(H1)(msg1, dst2, z1)
  y1 = _node1_call(p1[:N], p1[N_PAD:N_PAD + N], x, root1,
                   bias1.reshape(1, H1))

  ys = _make_gather(H1)(y1, src_p)
  W2a_p = jnp.pad(W2a, ((0, 0), (0, 112)))
  b2a_p = jnp.pad(b2a, (0, 112)).reshape(1, 128)
  msg2 = _msg_call(ys, edge_attr, W2a_p, b2a_p, M2, _R2, _U2, _S2, H2)
  p2 = _make_scatter(H2)(msg2, dst2, z2)
  out = _node2_call(p2[:N], p2[N_PAD:N_PAD + N], y1, root2,
                    bias2.reshape(1, H2), lin1_w, lin1_b.reshape(1, 8),
                    lin2_w, lin2_b.reshape(1, 1))
  return out
